# baseline (speedup 1.0000x reference)
import functools

import jax
import jax.numpy as jnp
from jax import lax
from jax.experimental import pallas as pl
from jax.experimental.pallas import tpu as pltpu
from jax.experimental.pallas import tpu_sc as plsc

VOCAB = 100000
EMBED = 128
CTX = 4096
B = 4
T = 4096

_info = plsc.get_sparse_core_info()
NC, NS, L = _info.num_cores, _info.num_subcores, _info.num_lanes
NW = NC * NS
TBLK = T // NW
SUB = 2
CROWS = TBLK // SUB
NCH = B * SUB


def _body(x_hbm, tok_hbm, pos_hbm, out_hbm,
          idx_v, tok_v, pos_v, sem_p, sem_o, *sems_g):
    wid = lax.axis_index("s") * NC + lax.axis_index("c")
    t0 = wid * TBLK

    i_descs = [
        pltpu.async_copy(x_hbm.at[b, pl.ds(t0, TBLK)], idx_v.at[b],
                         sems_g[NCH + b])
        for b in range(B)
    ]
    p_desc = pltpu.async_copy(pos_hbm.at[pl.ds(t0, TBLK)], pos_v, sem_p)

    g = [None] * NCH
    for k in range(NCH):
        b, off = k // SUB, (k % SUB) * CROWS
        if off == 0:
            i_descs[b].wait()
        g[k] = pltpu.async_copy(
            tok_hbm.at[idx_v.at[b, pl.ds(off, CROWS)]],
            tok_v.at[k], sems_g[k])

    o = [None] * NCH
    for k in range(NCH):
        b, off = k // SUB, (k % SUB) * CROWS
        g[k].wait()
        if k == 0:
            p_desc.wait()

        @plsc.parallel_loop(0, CROWS, unroll=4)
        def add_row(r):
            for j in range(EMBED // L):
                d = pl.ds(j * L, L)
                plsc.addupdate(tok_v.at[k, r, d], pos_v[off + r, d])

        o[k] = pltpu.async_copy(tok_v.at[k],
                                out_hbm.at[b, pl.ds(t0 + off, CROWS)],
                                sem_o)
    for k in range(NCH):
        o[k].wait()


_mesh = plsc.VectorSubcoreMesh(core_axis_name="c", subcore_axis_name="s")

_sc_call = functools.partial(
    pl.kernel,
    out_type=jax.ShapeDtypeStruct((B, T, EMBED), jnp.float32),
    mesh=_mesh,
    scratch_types=[
        pltpu.VMEM((B, TBLK), jnp.int32),
        pltpu.VMEM((NCH, CROWS, EMBED), jnp.float32),
        pltpu.VMEM((TBLK, EMBED), jnp.float32),
        pltpu.SemaphoreType.DMA,
        pltpu.SemaphoreType.DMA,
    ] + [pltpu.SemaphoreType.DMA] * (NCH + B),
)(_body)


def kernel(x, tok_table, pos_table):
    return _sc_call(x.astype(jnp.int32), tok_table, pos_table)
